# baseline (speedup 1.0000x reference)
import functools

import jax
import jax.numpy as jnp
from jax import lax
from jax.experimental import pallas as pl
from jax.experimental.pallas import tpu as pltpu
from jax.experimental.pallas import tpu_sc as plsc

L = 200
B = 1024
V = 1000000
D = 64
N = L * B

_info = plsc.get_sparse_core_info()
NC = _info.num_cores
NS = _info.num_subcores
NW = NC * NS
ROWS_W = N // NW
CHUNK = 400
NCH = ROWS_W // CHUNK
NBUF = 3


def _gather_body(embs_hbm, i0_hbm, i1_hbm, i2_hbm,
                 o0_hbm, o1_hbm, o2_hbm,
                 idx_v, bufs, gsems, ssems):
    wid = lax.axis_index("s") * NC + lax.axis_index("c")
    base = wid * ROWS_W

    def issue_gather(m, b):
        src = embs_hbm.at[idx_v.at[pl.ds(m * CHUNK, CHUNK)]]
        pltpu.make_async_copy(src, bufs.at[b], gsems.at[b]).start()

    def wait_gather(b):
        src = embs_hbm.at[idx_v.at[pl.ds(0, CHUNK)]]
        pltpu.make_async_copy(src, bufs.at[b], gsems.at[b]).wait()

    def issue_store(o_hbm, m, b):
        dst = o_hbm.at[pl.ds(base + m * CHUNK, CHUNK), :]
        pltpu.make_async_copy(bufs.at[b], dst, ssems.at[b]).start()

    def wait_store(o_hbm, b):
        dst = o_hbm.at[pl.ds(0, CHUNK), :]
        pltpu.make_async_copy(bufs.at[b], dst, ssems.at[b]).wait()

    for idx_hbm, o_hbm in ((i0_hbm, o0_hbm), (i1_hbm, o1_hbm),
                           (i2_hbm, o2_hbm)):
        pltpu.sync_copy(idx_hbm.at[pl.ds(base, ROWS_W)], idx_v)

        for m in range(NBUF):
            issue_gather(m, m)

        for m in range(NCH):
            b = m % NBUF
            wait_gather(b)
            issue_store(o_hbm, m, b)
            if m + NBUF < NCH:
                wait_store(o_hbm, b)
                issue_gather(m + NBUF, b)

        for m in range(NCH - NBUF, NCH):
            wait_store(o_hbm, m % NBUF)


_mesh = plsc.VectorSubcoreMesh(core_axis_name="c", subcore_axis_name="s")

_gather3 = functools.partial(
    pl.kernel,
    out_type=(
        jax.ShapeDtypeStruct((N, D), jnp.float32),
        jax.ShapeDtypeStruct((N, D), jnp.float32),
        jax.ShapeDtypeStruct((N, D), jnp.float32),
    ),
    mesh=_mesh,
    compiler_params=pltpu.CompilerParams(use_tc_tiling_on_sc=False),
    scratch_types=[
        pltpu.VMEM((ROWS_W,), jnp.int32),
        pltpu.VMEM((NBUF, CHUNK, D), jnp.float32),
        pltpu.SemaphoreType.DMA((NBUF,)),
        pltpu.SemaphoreType.DMA((NBUF,)),
    ],
)(_gather_body)


def kernel(embs, idx0, idx1, idx2, p0, p1, p2, len0, len1, len2):
    o0, o1, o2 = _gather3(embs, idx0.reshape(N), idx1.reshape(N),
                          idx2.reshape(N))

    def unpack(o):
        return o.reshape(L, B, D)

    return (unpack(o1), p1, len1,
            unpack(o2), p2, len2,
            unpack(o0), p0, len0)


# speedup vs baseline: 1.0290x; 1.0290x over previous
import functools

import jax
import jax.numpy as jnp
from jax import lax
from jax.experimental import pallas as pl
from jax.experimental.pallas import tpu as pltpu
from jax.experimental.pallas import tpu_sc as plsc

L = 200
B = 1024
V = 1000000
D = 64
N = L * B

_info = plsc.get_sparse_core_info()
NC = _info.num_cores
NS = _info.num_subcores
NW = NC * NS
ROWS_W = N // NW
CHUNK = 400
NCH = ROWS_W // CHUNK
NBUF = 3


def _gather_body(embs_hbm, idx_hbm, o_hbm, idx_v, bufs, gsems, ssems):
    wid = lax.axis_index("s") * NC + lax.axis_index("c")
    base = wid * ROWS_W

    def issue_gather(m, b):
        src = embs_hbm.at[idx_v.at[pl.ds(m * CHUNK, CHUNK)]]
        pltpu.make_async_copy(src, bufs.at[b], gsems.at[b]).start()

    def wait_gather(b):
        src = embs_hbm.at[idx_v.at[pl.ds(0, CHUNK)]]
        pltpu.make_async_copy(src, bufs.at[b], gsems.at[b]).wait()

    def issue_store(m, b):
        dst = o_hbm.at[pl.ds(base + m * CHUNK, CHUNK), :]
        pltpu.make_async_copy(bufs.at[b], dst, ssems.at[b]).start()

    def wait_store(b):
        dst = o_hbm.at[pl.ds(0, CHUNK), :]
        pltpu.make_async_copy(bufs.at[b], dst, ssems.at[b]).wait()

    pltpu.sync_copy(idx_hbm.at[pl.ds(base, ROWS_W)], idx_v)

    for m in range(NBUF):
        issue_gather(m, m)

    for m in range(NCH):
        b = m % NBUF
        wait_gather(b)
        issue_store(m, b)
        if m + NBUF < NCH:
            wait_store(b)
            issue_gather(m + NBUF, b)

    for m in range(NCH - NBUF, NCH):
        wait_store(m % NBUF)


_mesh = plsc.VectorSubcoreMesh(core_axis_name="c", subcore_axis_name="s")

_gather1 = functools.partial(
    pl.kernel,
    out_type=jax.ShapeDtypeStruct((N, D), jnp.float32),
    mesh=_mesh,
    compiler_params=pltpu.CompilerParams(use_tc_tiling_on_sc=False),
    scratch_types=[
        pltpu.VMEM((ROWS_W,), jnp.int32),
        pltpu.VMEM((NBUF, CHUNK, D), jnp.float32),
        pltpu.SemaphoreType.DMA((NBUF,)),
        pltpu.SemaphoreType.DMA((NBUF,)),
    ],
)(_gather_body)


def kernel(embs, idx0, idx1, idx2, p0, p1, p2, len0, len1, len2):
    o0 = _gather1(embs, idx0.reshape(N))
    o1 = _gather1(embs, idx1.reshape(N))
    o2 = _gather1(embs, idx2.reshape(N))

    def unpack(o):
        return o.reshape(L, B, D)

    return (unpack(o1), p1, len1,
            unpack(o2), p2, len2,
            unpack(o0), p0, len0)
